# Initial kernel scaffold; baseline (speedup 1.0000x reference)
import jax
import jax.numpy as jnp
from jax.experimental import pallas as pl


def kernel(vertices, faces):
    raise NotImplementedError("write your pallas kernel here")


# speedup vs baseline: 23.8566x; 23.8566x over previous
import functools

import jax
import jax.numpy as jnp
import numpy as np
from jax import lax
from jax.experimental import pallas as pl
from jax.experimental.pallas import tpu as pltpu

_S = 500
_FB = 512
_ROWS = 32
_COLS = 128
_CHUNK = 512


def _geom(vertices, faces):
    elev = 30.0 * np.pi / 180.0
    azim = 0.0
    rot_y = jnp.array([[np.cos(azim), 0.0, np.sin(azim)],
                       [0.0, 1.0, 0.0],
                       [-np.sin(azim), 0.0, np.cos(azim)]], dtype=jnp.float32)
    rot_x = jnp.array([[1.0, 0.0, 0.0],
                       [0.0, np.cos(elev), -np.sin(elev)],
                       [0.0, np.sin(elev), np.cos(elev)]], dtype=jnp.float32)
    vr = vertices @ (rot_y @ rot_x)
    vr = vr.at[:, 2].add(2.0)
    v2d = vr[:, :2] / vr[:, 2:3]
    v2d = (v2d + 1.0) * _S / 2.0
    tris = v2d[faces]
    v0 = vr[faces[:, 0]]
    v1 = vr[faces[:, 1]]
    v2 = vr[faces[:, 2]]
    n = jnp.cross(v1 - v0, v2 - v0)
    n = n / (jnp.linalg.norm(n, axis=1, keepdims=True) + 1e-08)
    light = jnp.array([0.0, 0.0, 1.0], dtype=jnp.float32)
    diffuse = jnp.clip(n @ light, 0.0, 1.0) * 180.0 + 75.0
    z_depths = vr[faces, 2].mean(axis=1)
    tri = tris.astype(jnp.int32)
    return tri, diffuse, z_depths


def _raster_body(tri_ref, fpar_ref, out_ref, zbuf, ibuf):
    g = pl.program_id(0)

    @pl.when(g == 0)
    def _init():
        zbuf[...] = jnp.full((_FB, _FB), jnp.inf, jnp.float32)
        ibuf[...] = jnp.full((_FB, _FB), -1, jnp.int32)
        out_ref[...] = jnp.full((_FB, _FB), 255.0, jnp.float32)

    def face_body(i, carry):
        x0 = tri_ref[0, 0, i]
        y0 = tri_ref[0, 1, i]
        x1 = tri_ref[0, 2, i]
        y1 = tri_ref[0, 3, i]
        x2 = tri_ref[0, 4, i]
        y2 = tri_ref[0, 5, i]
        z = fpar_ref[0, 0, i]
        col = fpar_ref[0, 1, i]
        c = (x1 - x0) * (y2 - y0) - (x2 - x0) * (y1 - y0)
        minx = jnp.maximum(0, jnp.minimum(jnp.minimum(x0, x1), x2))
        maxx = jnp.minimum(_S - 1, jnp.maximum(jnp.maximum(x0, x1), x2) + 1)
        miny = jnp.maximum(0, jnp.minimum(jnp.minimum(y0, y1), y2))
        maxy = jnp.minimum(_S - 1, jnp.maximum(jnp.maximum(y0, y1), y2) + 1)
        ok = (c != 0) & (maxx > minx) & (maxy > miny)

        @pl.when(ok)
        def _raster():
            gi = g * _CHUNK + i
            r_lo = (miny // _ROWS) * _ROWS
            n_rt = (maxy - 1) // _ROWS - miny // _ROWS + 1
            c_lo = (minx // _COLS) * _COLS
            n_ct = (maxx - 1) // _COLS - minx // _COLS + 1
            abs_c = jnp.abs(c)

            def rt_body(rt, carry_r):
                rr = r_lo + rt * _ROWS

                def ct_body(ct, carry_c):
                    cc = c_lo + ct * _COLS
                    xs = cc + lax.broadcasted_iota(jnp.int32, (_ROWS, _COLS), 1)
                    ys = rr + lax.broadcasted_iota(jnp.int32, (_ROWS, _COLS), 0)
                    a0 = jnp.abs((x1 - xs) * (y2 - ys) - (x2 - xs) * (y1 - ys))
                    a1 = jnp.abs((x2 - xs) * (y0 - ys) - (x0 - xs) * (y2 - ys))
                    inside = (a0 + a1) <= abs_c
                    inb = ((xs >= minx) & (xs < maxx)
                           & (ys >= miny) & (ys < maxy))
                    rows = pl.ds(rr, _ROWS)
                    cols = pl.ds(cc, _COLS)
                    zb = zbuf[rows, cols]
                    ib = ibuf[rows, cols]
                    win = inb & inside & ((z < zb) | ((z == zb) & (gi > ib)))
                    zbuf[rows, cols] = jnp.where(win, z, zb)
                    ibuf[rows, cols] = jnp.where(win, gi, ib)
                    cb = out_ref[rows, cols]
                    out_ref[rows, cols] = jnp.where(win, col, cb)
                    return carry_c

                return lax.fori_loop(0, n_ct, ct_body, carry_r)

            lax.fori_loop(0, n_rt, rt_body, 0)

        return carry

    lax.fori_loop(0, _CHUNK, face_body, 0)


@jax.jit
def kernel(vertices, faces):
    n_faces = faces.shape[0]
    tri, diffuse, z_depths = _geom(vertices, faces)

    n_grid = (n_faces + _CHUNK - 1) // _CHUNK
    n_pad = n_grid * _CHUNK
    tri_flat = jnp.zeros((n_pad, 6), jnp.int32)
    tri_flat = tri_flat.at[:n_faces].set(tri.reshape(n_faces, 6))
    fpar = jnp.zeros((n_pad, 2), jnp.float32)
    fpar = fpar.at[:n_faces, 0].set(z_depths)
    fpar = fpar.at[:n_faces, 1].set(diffuse)
    tri_blocks = tri_flat.reshape(n_grid, _CHUNK, 6).transpose(0, 2, 1)
    fpar_blocks = fpar.reshape(n_grid, _CHUNK, 2).transpose(0, 2, 1)

    color = pl.pallas_call(
        _raster_body,
        grid=(n_grid,),
        in_specs=[
            pl.BlockSpec((1, 6, _CHUNK), lambda g: (g, 0, 0),
                         memory_space=pltpu.SMEM),
            pl.BlockSpec((1, 2, _CHUNK), lambda g: (g, 0, 0),
                         memory_space=pltpu.SMEM),
        ],
        out_specs=pl.BlockSpec((_FB, _FB), lambda g: (0, 0)),
        out_shape=jax.ShapeDtypeStruct((_FB, _FB), jnp.float32),
        scratch_shapes=[
            pltpu.VMEM((_FB, _FB), jnp.float32),
            pltpu.VMEM((_FB, _FB), jnp.int32),
        ],
    )(tri_blocks, fpar_blocks)

    return jnp.broadcast_to(color[:_S, :_S, None], (_S, _S, 3))


# speedup vs baseline: 24.8026x; 1.0397x over previous
import functools

import jax
import jax.numpy as jnp
import numpy as np
from jax import lax
from jax.experimental import pallas as pl
from jax.experimental.pallas import tpu as pltpu

_S = 500
_FB = 512
_ROWS = 32
_COLS = 128
_CHUNK = 512


def _geom(vertices, faces):
    elev = 30.0 * np.pi / 180.0
    azim = 0.0
    rot_y = jnp.array([[np.cos(azim), 0.0, np.sin(azim)],
                       [0.0, 1.0, 0.0],
                       [-np.sin(azim), 0.0, np.cos(azim)]], dtype=jnp.float32)
    rot_x = jnp.array([[1.0, 0.0, 0.0],
                       [0.0, np.cos(elev), -np.sin(elev)],
                       [0.0, np.sin(elev), np.cos(elev)]], dtype=jnp.float32)
    vr = vertices @ (rot_y @ rot_x)
    vr = vr.at[:, 2].add(2.0)
    v2d = vr[:, :2] / vr[:, 2:3]
    v2d = (v2d + 1.0) * _S / 2.0
    tris = v2d[faces]
    v0 = vr[faces[:, 0]]
    v1 = vr[faces[:, 1]]
    v2 = vr[faces[:, 2]]
    n = jnp.cross(v1 - v0, v2 - v0)
    n = n / (jnp.linalg.norm(n, axis=1, keepdims=True) + 1e-08)
    light = jnp.array([0.0, 0.0, 1.0], dtype=jnp.float32)
    diffuse = jnp.clip(n @ light, 0.0, 1.0) * 180.0 + 75.0
    z_depths = vr[faces, 2].mean(axis=1)
    tri = tris.astype(jnp.int32)
    return tri, diffuse, z_depths


def _raster_body(tri_ref, fpar_ref, out_ref, zbuf):
    g = pl.program_id(0)

    @pl.when(g == 0)
    def _init():
        zbuf[...] = jnp.full((_FB, _FB), jnp.inf, jnp.float32)
        out_ref[...] = jnp.full((_FB, _FB), 255.0, jnp.float32)

    def face_body(i, carry):
        x0 = tri_ref[0, 0, i]
        y0 = tri_ref[0, 1, i]
        x1 = tri_ref[0, 2, i]
        y1 = tri_ref[0, 3, i]
        x2 = tri_ref[0, 4, i]
        y2 = tri_ref[0, 5, i]
        z = fpar_ref[0, 0, i]
        col = fpar_ref[0, 1, i]
        c = (x1 - x0) * (y2 - y0) - (x2 - x0) * (y1 - y0)
        minx = jnp.maximum(0, jnp.minimum(jnp.minimum(x0, x1), x2))
        maxx = jnp.minimum(_S - 1, jnp.maximum(jnp.maximum(x0, x1), x2) + 1)
        miny = jnp.maximum(0, jnp.minimum(jnp.minimum(y0, y1), y2))
        maxy = jnp.minimum(_S - 1, jnp.maximum(jnp.maximum(y0, y1), y2) + 1)
        ok = (c != 0) & (maxx > minx) & (maxy > miny)

        @pl.when(ok)
        def _raster():
            r_lo = (miny // _ROWS) * _ROWS
            n_rt = (maxy - 1) // _ROWS - miny // _ROWS + 1
            c_lo = (minx // _COLS) * _COLS
            n_ct = (maxx - 1) // _COLS - minx // _COLS + 1
            abs_c = jnp.abs(c)

            def rt_body(rt, carry_r):
                rr = r_lo + rt * _ROWS

                def ct_body(ct, carry_c):
                    cc = c_lo + ct * _COLS
                    xs = cc + lax.broadcasted_iota(jnp.int32, (_ROWS, _COLS), 1)
                    ys = rr + lax.broadcasted_iota(jnp.int32, (_ROWS, _COLS), 0)
                    a0 = jnp.abs((x1 - xs) * (y2 - ys) - (x2 - xs) * (y1 - ys))
                    a1 = jnp.abs((x2 - xs) * (y0 - ys) - (x0 - xs) * (y2 - ys))
                    inside = (a0 + a1) <= abs_c
                    inb = ((xs >= minx) & (xs < maxx)
                           & (ys >= miny) & (ys < maxy))
                    rows = pl.ds(rr, _ROWS)
                    cols = pl.ds(cc, _COLS)
                    zb = zbuf[rows, cols]
                    win = inb & inside & (z <= zb)
                    zbuf[rows, cols] = jnp.where(win, z, zb)
                    cb = out_ref[rows, cols]
                    out_ref[rows, cols] = jnp.where(win, col, cb)
                    return carry_c

                return lax.fori_loop(0, n_ct, ct_body, carry_r)

            lax.fori_loop(0, n_rt, rt_body, 0)

        return carry

    lax.fori_loop(0, _CHUNK, face_body, 0)


@jax.jit
def kernel(vertices, faces):
    n_faces = faces.shape[0]
    tri, diffuse, z_depths = _geom(vertices, faces)

    n_grid = (n_faces + _CHUNK - 1) // _CHUNK
    n_pad = n_grid * _CHUNK
    tri_flat = jnp.zeros((n_pad, 6), jnp.int32)
    tri_flat = tri_flat.at[:n_faces].set(tri.reshape(n_faces, 6))
    fpar = jnp.zeros((n_pad, 2), jnp.float32)
    fpar = fpar.at[:n_faces, 0].set(z_depths)
    fpar = fpar.at[:n_faces, 1].set(diffuse)
    tri_blocks = tri_flat.reshape(n_grid, _CHUNK, 6).transpose(0, 2, 1)
    fpar_blocks = fpar.reshape(n_grid, _CHUNK, 2).transpose(0, 2, 1)

    color = pl.pallas_call(
        _raster_body,
        grid=(n_grid,),
        in_specs=[
            pl.BlockSpec((1, 6, _CHUNK), lambda g: (g, 0, 0),
                         memory_space=pltpu.SMEM),
            pl.BlockSpec((1, 2, _CHUNK), lambda g: (g, 0, 0),
                         memory_space=pltpu.SMEM),
        ],
        out_specs=pl.BlockSpec((_FB, _FB), lambda g: (0, 0)),
        out_shape=jax.ShapeDtypeStruct((_FB, _FB), jnp.float32),
        scratch_shapes=[
            pltpu.VMEM((_FB, _FB), jnp.float32),
        ],
    )(tri_blocks, fpar_blocks)

    return jnp.broadcast_to(color[:_S, :_S, None], (_S, _S, 3))


# speedup vs baseline: 29.4030x; 1.1855x over previous
import functools

import jax
import jax.numpy as jnp
import numpy as np
from jax import lax
from jax.experimental import pallas as pl
from jax.experimental.pallas import tpu as pltpu

_S = 500
_FB = 512
_ROWS = 32
_COLS = 128
_CHUNK = 512


def _geom(vertices, faces):
    elev = 30.0 * np.pi / 180.0
    azim = 0.0
    rot_y = jnp.array([[np.cos(azim), 0.0, np.sin(azim)],
                       [0.0, 1.0, 0.0],
                       [-np.sin(azim), 0.0, np.cos(azim)]], dtype=jnp.float32)
    rot_x = jnp.array([[1.0, 0.0, 0.0],
                       [0.0, np.cos(elev), -np.sin(elev)],
                       [0.0, np.sin(elev), np.cos(elev)]], dtype=jnp.float32)
    vr = vertices @ (rot_y @ rot_x)
    vr = vr.at[:, 2].add(2.0)
    v2d = vr[:, :2] / vr[:, 2:3]
    v2d = (v2d + 1.0) * _S / 2.0
    tris = v2d[faces]
    v0 = vr[faces[:, 0]]
    v1 = vr[faces[:, 1]]
    v2 = vr[faces[:, 2]]
    n = jnp.cross(v1 - v0, v2 - v0)
    n = n / (jnp.linalg.norm(n, axis=1, keepdims=True) + 1e-08)
    light = jnp.array([0.0, 0.0, 1.0], dtype=jnp.float32)
    diffuse = jnp.clip(n @ light, 0.0, 1.0) * 180.0 + 75.0
    z_depths = vr[faces, 2].mean(axis=1)
    tri = tris.astype(jnp.int32)
    return tri, diffuse, z_depths


def _raster_body(tri_ref, fpar_ref, out_ref, zbuf):
    g = pl.program_id(0)

    @pl.when(g == 0)
    def _init():
        zbuf[...] = jnp.full((_FB, _FB), jnp.inf, jnp.float32)
        out_ref[...] = jnp.full((_FB, _FB), 255.0, jnp.float32)

    def face_body(i, carry):
        x0 = tri_ref[0, 0, i]
        y0 = tri_ref[0, 1, i]
        x1 = tri_ref[0, 2, i]
        y1 = tri_ref[0, 3, i]
        x2 = tri_ref[0, 4, i]
        y2 = tri_ref[0, 5, i]
        abs_c = tri_ref[0, 6, i]
        minx = tri_ref[0, 7, i]
        maxx = tri_ref[0, 8, i]
        miny = tri_ref[0, 9, i]
        maxy = tri_ref[0, 10, i]
        r_lo = tri_ref[0, 11, i]
        c_lo = tri_ref[0, 12, i]
        c_hi = tri_ref[0, 13, i]
        total = tri_ref[0, 14, i]
        z = fpar_ref[0, 0, i]
        col = fpar_ref[0, 1, i]

        def win_body(st, rc):
            rr, cc = rc
            xs = cc + lax.broadcasted_iota(jnp.int32, (_ROWS, _COLS), 1)
            ys = rr + lax.broadcasted_iota(jnp.int32, (_ROWS, _COLS), 0)
            a0 = jnp.abs((x1 - xs) * (y2 - ys) - (x2 - xs) * (y1 - ys))
            a1 = jnp.abs((x2 - xs) * (y0 - ys) - (x0 - xs) * (y2 - ys))
            inside = (a0 + a1) <= abs_c
            inb = ((xs >= minx) & (xs < maxx)
                   & (ys >= miny) & (ys < maxy))
            rows = pl.ds(pl.multiple_of(rr, _ROWS), _ROWS)
            cols = pl.ds(pl.multiple_of(cc, _COLS), _COLS)
            zb = zbuf[rows, cols]
            win = inb & inside & (z <= zb)
            zbuf[rows, cols] = jnp.where(win, z, zb)
            cb = out_ref[rows, cols]
            out_ref[rows, cols] = jnp.where(win, col, cb)
            cc2 = cc + _COLS
            wrap = cc2 >= c_hi
            return (jnp.where(wrap, rr + _ROWS, rr),
                    jnp.where(wrap, c_lo, cc2))

        lax.fori_loop(0, total, win_body, (r_lo, c_lo))
        return carry

    lax.fori_loop(0, _CHUNK, face_body, 0)


@jax.jit
def kernel(vertices, faces):
    n_faces = faces.shape[0]
    tri, diffuse, z_depths = _geom(vertices, faces)

    x0, y0 = tri[:, 0, 0], tri[:, 0, 1]
    x1, y1 = tri[:, 1, 0], tri[:, 1, 1]
    x2, y2 = tri[:, 2, 0], tri[:, 2, 1]
    c = (x1 - x0) * (y2 - y0) - (x2 - x0) * (y1 - y0)
    minx = jnp.maximum(0, jnp.minimum(jnp.minimum(x0, x1), x2))
    maxx = jnp.minimum(_S - 1, jnp.maximum(jnp.maximum(x0, x1), x2) + 1)
    miny = jnp.maximum(0, jnp.minimum(jnp.minimum(y0, y1), y2))
    maxy = jnp.minimum(_S - 1, jnp.maximum(jnp.maximum(y0, y1), y2) + 1)
    ok = (c != 0) & (maxx > minx) & (maxy > miny)
    r_lo = (miny // _ROWS) * _ROWS
    n_rt = (maxy - 1) // _ROWS - miny // _ROWS + 1
    c_lo = (minx // _COLS) * _COLS
    n_ct = (maxx - 1) // _COLS - minx // _COLS + 1
    c_hi = c_lo + n_ct * _COLS
    total = jnp.where(ok, n_rt * n_ct, 0)
    params = jnp.stack(
        [x0, y0, x1, y1, x2, y2, jnp.abs(c), minx, maxx, miny, maxy,
         r_lo, c_lo, c_hi, total, jnp.zeros_like(c)], axis=1)

    n_grid = (n_faces + _CHUNK - 1) // _CHUNK
    n_pad = n_grid * _CHUNK
    tri_flat = jnp.zeros((n_pad, 16), jnp.int32)
    tri_flat = tri_flat.at[:n_faces].set(params)
    fpar = jnp.zeros((n_pad, 2), jnp.float32)
    fpar = fpar.at[:n_faces, 0].set(z_depths)
    fpar = fpar.at[:n_faces, 1].set(diffuse)
    tri_blocks = tri_flat.reshape(n_grid, _CHUNK, 16).transpose(0, 2, 1)
    fpar_blocks = fpar.reshape(n_grid, _CHUNK, 2).transpose(0, 2, 1)

    color = pl.pallas_call(
        _raster_body,
        grid=(n_grid,),
        in_specs=[
            pl.BlockSpec((1, 16, _CHUNK), lambda g: (g, 0, 0),
                         memory_space=pltpu.SMEM),
            pl.BlockSpec((1, 2, _CHUNK), lambda g: (g, 0, 0),
                         memory_space=pltpu.SMEM),
        ],
        out_specs=pl.BlockSpec((_FB, _FB), lambda g: (0, 0)),
        out_shape=jax.ShapeDtypeStruct((_FB, _FB), jnp.float32),
        scratch_shapes=[
            pltpu.VMEM((_FB, _FB), jnp.float32),
        ],
    )(tri_blocks, fpar_blocks)

    return jnp.broadcast_to(color[:_S, :_S, None], (_S, _S, 3))


# speedup vs baseline: 50.9771x; 1.7337x over previous
import functools

import jax
import jax.numpy as jnp
import numpy as np
from jax import lax
from jax.experimental import pallas as pl
from jax.experimental.pallas import tpu as pltpu
from jax.experimental.pallas import tpu_sc as plsc

_S = 500
_FB = 512
_ROWS = 32
_COLS = 128
_CHUNK = 512
_NV = 20000
_NPAD = 51200
_NW = 32
_NCH = 5
_FCH = 320


def _sc_face_params(xi, yi, vrx, vry, vrz, f0, f1, f2):
    mesh = plsc.VectorSubcoreMesh(core_axis_name="c", subcore_axis_name="s",
                                  num_cores=2)

    @functools.partial(
        pl.kernel, mesh=mesh,
        compiler_params=pltpu.CompilerParams(needs_layout_passes=False),
        out_type=[
            jax.ShapeDtypeStruct((_NW, _NCH, 16, _FCH), jnp.int32),
            jax.ShapeDtypeStruct((_NW, _NCH, 2, _FCH), jnp.float32),
        ],
        scratch_types=[
            pltpu.VMEM((_NV,), jnp.int32),
            pltpu.VMEM((_NV,), jnp.int32),
            pltpu.VMEM((_NV,), jnp.float32),
            pltpu.VMEM((_NV,), jnp.float32),
            pltpu.VMEM((_NV,), jnp.float32),
            pltpu.VMEM((_FCH,), jnp.int32),
            pltpu.VMEM((_FCH,), jnp.int32),
            pltpu.VMEM((_FCH,), jnp.int32),
            pltpu.VMEM((16, _FCH), jnp.int32),
            pltpu.VMEM((2, _FCH), jnp.float32),
        ],
    )
    def k(xi_h, yi_h, vrx_h, vry_h, vrz_h, f0_h, f1_h, f2_h, pi_h, pf_h,
          xi_v, yi_v, vrx_v, vry_v, vrz_v, i0_v, i1_v, i2_v, oi_v, of_v):
        wid = lax.axis_index("s") * 2 + lax.axis_index("c")
        pltpu.sync_copy(xi_h, xi_v)
        pltpu.sync_copy(yi_h, yi_v)
        pltpu.sync_copy(vrx_h, vrx_v)
        pltpu.sync_copy(vry_h, vry_v)
        pltpu.sync_copy(vrz_h, vrz_v)
        for ch in range(_NCH):
            base = wid * (_NCH * _FCH) + ch * _FCH
            pltpu.sync_copy(f0_h.at[pl.ds(base, _FCH)], i0_v)
            pltpu.sync_copy(f1_h.at[pl.ds(base, _FCH)], i1_v)
            pltpu.sync_copy(f2_h.at[pl.ds(base, _FCH)], i2_v)

            def body(t, carry):
                sl = pl.ds(t * 16, 16)
                i0 = i0_v[sl]
                i1 = i1_v[sl]
                i2 = i2_v[sl]
                x0 = plsc.load_gather(xi_v, [i0])
                y0 = plsc.load_gather(yi_v, [i0])
                x1 = plsc.load_gather(xi_v, [i1])
                y1 = plsc.load_gather(yi_v, [i1])
                x2 = plsc.load_gather(xi_v, [i2])
                y2 = plsc.load_gather(yi_v, [i2])
                ax = plsc.load_gather(vrx_v, [i0])
                ay = plsc.load_gather(vry_v, [i0])
                az = plsc.load_gather(vrz_v, [i0])
                bx = plsc.load_gather(vrx_v, [i1])
                by = plsc.load_gather(vry_v, [i1])
                bz = plsc.load_gather(vrz_v, [i1])
                cx = plsc.load_gather(vrx_v, [i2])
                cy = plsc.load_gather(vry_v, [i2])
                cz = plsc.load_gather(vrz_v, [i2])
                c = (x1 - x0) * (y2 - y0) - (x2 - x0) * (y1 - y0)
                minx = jnp.maximum(0, jnp.minimum(jnp.minimum(x0, x1), x2))
                maxx = jnp.minimum(
                    _S - 1, jnp.maximum(jnp.maximum(x0, x1), x2) + 1)
                miny = jnp.maximum(0, jnp.minimum(jnp.minimum(y0, y1), y2))
                maxy = jnp.minimum(
                    _S - 1, jnp.maximum(jnp.maximum(y0, y1), y2) + 1)
                ok = (c != 0) & (maxx > minx) & (maxy > miny)
                rt0 = lax.shift_right_arithmetic(miny, 5)
                r_lo = lax.shift_left(rt0, 5)
                n_rt = lax.shift_right_arithmetic(maxy - 1, 5) - rt0 + 1
                ct0 = lax.shift_right_arithmetic(minx, 7)
                c_lo = lax.shift_left(ct0, 7)
                n_ct = lax.shift_right_arithmetic(maxx - 1, 7) - ct0 + 1
                c_hi = c_lo + lax.shift_left(n_ct, 7)
                total = jnp.where(ok, n_rt * n_ct, 0)
                z = ((az + bz) + cz) / 3.0
                ux, uy, uz = bx - ax, by - ay, bz - az
                wx, wy, wz = cx - ax, cy - ay, cz - az
                nx = uy * wz - uz * wy
                ny = uz * wx - ux * wz
                nz = ux * wy - uy * wx
                s = nx * nx + ny * ny + nz * nz
                h = jnp.float32(0.5) * s
                yv = plsc.bitcast(
                    jnp.int32(0x5F3759DF)
                    - lax.shift_right_arithmetic(plsc.bitcast(s, jnp.int32), 1),
                    jnp.float32)
                for _ in range(3):
                    yv = yv * (jnp.float32(1.5) - h * yv * yv)
                norm = jnp.where(s > 0, s * yv, jnp.float32(0.0))
                d = jnp.clip(nz / (norm + jnp.float32(1e-8)), 0.0, 1.0)
                col = d * jnp.float32(180.0) + jnp.float32(75.0)
                oi_v[0, sl] = x0
                oi_v[1, sl] = y0
                oi_v[2, sl] = x1
                oi_v[3, sl] = y1
                oi_v[4, sl] = x2
                oi_v[5, sl] = y2
                oi_v[6, sl] = jnp.abs(c)
                oi_v[7, sl] = minx
                oi_v[8, sl] = maxx
                oi_v[9, sl] = miny
                oi_v[10, sl] = maxy
                oi_v[11, sl] = r_lo
                oi_v[12, sl] = c_lo
                oi_v[13, sl] = c_hi
                oi_v[14, sl] = total
                oi_v[15, sl] = jnp.zeros((16,), jnp.int32)
                of_v[0, sl] = z
                of_v[1, sl] = col
                return carry

            lax.fori_loop(0, _FCH // 16, body, 0)
            pltpu.sync_copy(oi_v, pi_h.at[wid, ch])
            pltpu.sync_copy(of_v, pf_h.at[wid, ch])

    return k(xi, yi, vrx, vry, vrz, f0, f1, f2)


def _raster_body(tri_ref, fpar_ref, out_ref, zbuf):
    g = pl.program_id(0)

    @pl.when(g == 0)
    def _init():
        zbuf[...] = jnp.full((_FB, _FB), jnp.inf, jnp.float32)
        out_ref[...] = jnp.full((_FB, _FB), 255.0, jnp.float32)

    def face_body(i, carry):
        x0 = tri_ref[0, 0, i]
        y0 = tri_ref[0, 1, i]
        x1 = tri_ref[0, 2, i]
        y1 = tri_ref[0, 3, i]
        x2 = tri_ref[0, 4, i]
        y2 = tri_ref[0, 5, i]
        abs_c = tri_ref[0, 6, i]
        minx = tri_ref[0, 7, i]
        maxx = tri_ref[0, 8, i]
        miny = tri_ref[0, 9, i]
        maxy = tri_ref[0, 10, i]
        r_lo = tri_ref[0, 11, i]
        c_lo = tri_ref[0, 12, i]
        c_hi = tri_ref[0, 13, i]
        total = tri_ref[0, 14, i]
        z = fpar_ref[0, 0, i]
        col = fpar_ref[0, 1, i]

        def win_body(st, rc):
            rr, cc = rc
            xs = cc + lax.broadcasted_iota(jnp.int32, (_ROWS, _COLS), 1)
            ys = rr + lax.broadcasted_iota(jnp.int32, (_ROWS, _COLS), 0)
            a0 = jnp.abs((x1 - xs) * (y2 - ys) - (x2 - xs) * (y1 - ys))
            a1 = jnp.abs((x2 - xs) * (y0 - ys) - (x0 - xs) * (y2 - ys))
            inside = (a0 + a1) <= abs_c
            inb = ((xs >= minx) & (xs < maxx)
                   & (ys >= miny) & (ys < maxy))
            rows = pl.ds(pl.multiple_of(rr, _ROWS), _ROWS)
            cols = pl.ds(pl.multiple_of(cc, _COLS), _COLS)
            zb = zbuf[rows, cols]
            win = inb & inside & (z <= zb)
            zbuf[rows, cols] = jnp.where(win, z, zb)
            cb = out_ref[rows, cols]
            out_ref[rows, cols] = jnp.where(win, col, cb)
            cc2 = cc + _COLS
            wrap = cc2 >= c_hi
            return (jnp.where(wrap, rr + _ROWS, rr),
                    jnp.where(wrap, c_lo, cc2))

        lax.fori_loop(0, total, win_body, (r_lo, c_lo))
        return carry

    lax.fori_loop(0, _CHUNK, face_body, 0)


@jax.jit
def kernel(vertices, faces):
    n_faces = faces.shape[0]
    elev = 30.0 * np.pi / 180.0
    azim = 0.0
    rot_y = jnp.array([[np.cos(azim), 0.0, np.sin(azim)],
                       [0.0, 1.0, 0.0],
                       [-np.sin(azim), 0.0, np.cos(azim)]], dtype=jnp.float32)
    rot_x = jnp.array([[1.0, 0.0, 0.0],
                       [0.0, np.cos(elev), -np.sin(elev)],
                       [0.0, np.sin(elev), np.cos(elev)]], dtype=jnp.float32)
    vr = vertices @ (rot_y @ rot_x)
    vr = vr.at[:, 2].add(2.0)
    v2d = vr[:, :2] / vr[:, 2:3]
    v2d = (v2d + 1.0) * _S / 2.0
    xi = v2d[:, 0].astype(jnp.int32)
    yi = v2d[:, 1].astype(jnp.int32)
    vrx, vry, vrz = vr[:, 0], vr[:, 1], vr[:, 2]

    f_pad = jnp.zeros((_NPAD, 3), jnp.int32)
    f_pad = f_pad.at[:n_faces].set(faces.astype(jnp.int32))
    f0, f1, f2 = f_pad[:, 0], f_pad[:, 1], f_pad[:, 2]

    pi, pf = _sc_face_params(xi, yi, vrx, vry, vrz, f0, f1, f2)

    n_grid = _NPAD // _CHUNK
    tri_blocks = (pi.transpose(2, 0, 1, 3).reshape(16, _NPAD)
                  .reshape(16, n_grid, _CHUNK).transpose(1, 0, 2))
    fpar_blocks = (pf.transpose(2, 0, 1, 3).reshape(2, _NPAD)
                   .reshape(2, n_grid, _CHUNK).transpose(1, 0, 2))

    color = pl.pallas_call(
        _raster_body,
        grid=(n_grid,),
        in_specs=[
            pl.BlockSpec((1, 16, _CHUNK), lambda g: (g, 0, 0),
                         memory_space=pltpu.SMEM),
            pl.BlockSpec((1, 2, _CHUNK), lambda g: (g, 0, 0),
                         memory_space=pltpu.SMEM),
        ],
        out_specs=pl.BlockSpec((_FB, _FB), lambda g: (0, 0)),
        out_shape=jax.ShapeDtypeStruct((_FB, _FB), jnp.float32),
        scratch_shapes=[
            pltpu.VMEM((_FB, _FB), jnp.float32),
        ],
    )(tri_blocks, fpar_blocks)

    return jnp.broadcast_to(color[:_S, :_S, None], (_S, _S, 3))


# speedup vs baseline: 52.2572x; 1.0251x over previous
import functools

import jax
import jax.numpy as jnp
import numpy as np
from jax import lax
from jax.experimental import pallas as pl
from jax.experimental.pallas import tpu as pltpu
from jax.experimental.pallas import tpu_sc as plsc

_S = 500
_FB = 512
_ROWS = 32
_COLS = 128
_CHUNK = 512
_NV = 20000
_NPAD = 57344
_NW = 32
_NCH = 7
_FCH = 256


def _sc_face_params(xi, yi, vrx, vry, vrz, f0, f1, f2):
    mesh = plsc.VectorSubcoreMesh(core_axis_name="c", subcore_axis_name="s",
                                  num_cores=2)

    @functools.partial(
        pl.kernel, mesh=mesh,
        compiler_params=pltpu.CompilerParams(needs_layout_passes=False),
        out_type=[
            jax.ShapeDtypeStruct((16, _NPAD), jnp.int32),
            jax.ShapeDtypeStruct((2, _NPAD), jnp.float32),
        ],
        scratch_types=[
            pltpu.VMEM((_NV,), jnp.int32),
            pltpu.VMEM((_NV,), jnp.int32),
            pltpu.VMEM((_NV,), jnp.float32),
            pltpu.VMEM((_NV,), jnp.float32),
            pltpu.VMEM((_NV,), jnp.float32),
            pltpu.VMEM((_FCH,), jnp.int32),
            pltpu.VMEM((_FCH,), jnp.int32),
            pltpu.VMEM((_FCH,), jnp.int32),
            pltpu.VMEM((16, _FCH), jnp.int32),
            pltpu.VMEM((2, _FCH), jnp.float32),
        ],
    )
    def k(xi_h, yi_h, vrx_h, vry_h, vrz_h, f0_h, f1_h, f2_h, pi_h, pf_h,
          xi_v, yi_v, vrx_v, vry_v, vrz_v, i0_v, i1_v, i2_v, oi_v, of_v):
        wid = lax.axis_index("s") * 2 + lax.axis_index("c")
        pltpu.sync_copy(xi_h, xi_v)
        pltpu.sync_copy(yi_h, yi_v)
        pltpu.sync_copy(vrx_h, vrx_v)
        pltpu.sync_copy(vry_h, vry_v)
        pltpu.sync_copy(vrz_h, vrz_v)
        for ch in range(_NCH):
            base = wid * (_NCH * _FCH) + ch * _FCH
            pltpu.sync_copy(f0_h.at[pl.ds(base, _FCH)], i0_v)
            pltpu.sync_copy(f1_h.at[pl.ds(base, _FCH)], i1_v)
            pltpu.sync_copy(f2_h.at[pl.ds(base, _FCH)], i2_v)

            def body(t, carry):
                sl = pl.ds(t * 16, 16)
                i0 = i0_v[sl]
                i1 = i1_v[sl]
                i2 = i2_v[sl]
                x0 = plsc.load_gather(xi_v, [i0])
                y0 = plsc.load_gather(yi_v, [i0])
                x1 = plsc.load_gather(xi_v, [i1])
                y1 = plsc.load_gather(yi_v, [i1])
                x2 = plsc.load_gather(xi_v, [i2])
                y2 = plsc.load_gather(yi_v, [i2])
                ax = plsc.load_gather(vrx_v, [i0])
                ay = plsc.load_gather(vry_v, [i0])
                az = plsc.load_gather(vrz_v, [i0])
                bx = plsc.load_gather(vrx_v, [i1])
                by = plsc.load_gather(vry_v, [i1])
                bz = plsc.load_gather(vrz_v, [i1])
                cx = plsc.load_gather(vrx_v, [i2])
                cy = plsc.load_gather(vry_v, [i2])
                cz = plsc.load_gather(vrz_v, [i2])
                c = (x1 - x0) * (y2 - y0) - (x2 - x0) * (y1 - y0)
                minx = jnp.maximum(0, jnp.minimum(jnp.minimum(x0, x1), x2))
                maxx = jnp.minimum(
                    _S - 1, jnp.maximum(jnp.maximum(x0, x1), x2) + 1)
                miny = jnp.maximum(0, jnp.minimum(jnp.minimum(y0, y1), y2))
                maxy = jnp.minimum(
                    _S - 1, jnp.maximum(jnp.maximum(y0, y1), y2) + 1)
                ok = (c != 0) & (maxx > minx) & (maxy > miny)
                rt0 = lax.shift_right_arithmetic(miny, 5)
                r_lo = lax.shift_left(rt0, 5)
                n_rt = lax.shift_right_arithmetic(maxy - 1, 5) - rt0 + 1
                ct0 = lax.shift_right_arithmetic(minx, 7)
                c_lo = lax.shift_left(ct0, 7)
                n_ct = lax.shift_right_arithmetic(maxx - 1, 7) - ct0 + 1
                c_hi = c_lo + lax.shift_left(n_ct, 7)
                total = jnp.where(ok, n_rt * n_ct, 0)
                z = ((az + bz) + cz) / 3.0
                ux, uy, uz = bx - ax, by - ay, bz - az
                wx, wy, wz = cx - ax, cy - ay, cz - az
                nx = uy * wz - uz * wy
                ny = uz * wx - ux * wz
                nz = ux * wy - uy * wx
                s = nx * nx + ny * ny + nz * nz
                h = jnp.float32(0.5) * s
                yv = plsc.bitcast(
                    jnp.int32(0x5F3759DF)
                    - lax.shift_right_arithmetic(plsc.bitcast(s, jnp.int32), 1),
                    jnp.float32)
                for _ in range(3):
                    yv = yv * (jnp.float32(1.5) - h * yv * yv)
                norm = jnp.where(s > 0, s * yv, jnp.float32(0.0))
                d = jnp.clip(nz / (norm + jnp.float32(1e-8)), 0.0, 1.0)
                col = d * jnp.float32(180.0) + jnp.float32(75.0)
                oi_v[0, sl] = x0
                oi_v[1, sl] = y0
                oi_v[2, sl] = x1
                oi_v[3, sl] = y1
                oi_v[4, sl] = x2
                oi_v[5, sl] = y2
                oi_v[6, sl] = jnp.abs(c)
                oi_v[7, sl] = minx
                oi_v[8, sl] = maxx
                oi_v[9, sl] = miny
                oi_v[10, sl] = maxy
                oi_v[11, sl] = r_lo
                oi_v[12, sl] = c_lo
                oi_v[13, sl] = c_hi
                oi_v[14, sl] = total
                oi_v[15, sl] = jnp.zeros((16,), jnp.int32)
                of_v[0, sl] = z
                of_v[1, sl] = col
                return carry

            lax.fori_loop(0, _FCH // 16, body, 0)
            pltpu.sync_copy(oi_v, pi_h.at[:, pl.ds(base, _FCH)])
            pltpu.sync_copy(of_v, pf_h.at[:, pl.ds(base, _FCH)])

    return k(xi, yi, vrx, vry, vrz, f0, f1, f2)


def _raster_body(tri_ref, fpar_ref, out_ref, zbuf):
    g = pl.program_id(0)

    @pl.when(g == 0)
    def _init():
        zbuf[...] = jnp.full((_FB, _FB), jnp.inf, jnp.float32)
        out_ref[...] = jnp.full((_FB, _FB), 255.0, jnp.float32)

    def face_body(i, carry):
        x0 = tri_ref[0, 0, 0, i]
        y0 = tri_ref[1, 0, 0, i]
        x1 = tri_ref[2, 0, 0, i]
        y1 = tri_ref[3, 0, 0, i]
        x2 = tri_ref[4, 0, 0, i]
        y2 = tri_ref[5, 0, 0, i]
        abs_c = tri_ref[6, 0, 0, i]
        minx = tri_ref[7, 0, 0, i]
        maxx = tri_ref[8, 0, 0, i]
        miny = tri_ref[9, 0, 0, i]
        maxy = tri_ref[10, 0, 0, i]
        r_lo = tri_ref[11, 0, 0, i]
        c_lo = tri_ref[12, 0, 0, i]
        c_hi = tri_ref[13, 0, 0, i]
        total = tri_ref[14, 0, 0, i]
        z = fpar_ref[0, 0, 0, i]
        col = fpar_ref[1, 0, 0, i]

        def win_body(st, rc):
            rr, cc = rc
            xs = cc + lax.broadcasted_iota(jnp.int32, (_ROWS, _COLS), 1)
            ys = rr + lax.broadcasted_iota(jnp.int32, (_ROWS, _COLS), 0)
            a0 = jnp.abs((x1 - xs) * (y2 - ys) - (x2 - xs) * (y1 - ys))
            a1 = jnp.abs((x2 - xs) * (y0 - ys) - (x0 - xs) * (y2 - ys))
            inside = (a0 + a1) <= abs_c
            inb = ((xs >= minx) & (xs < maxx)
                   & (ys >= miny) & (ys < maxy))
            rows = pl.ds(pl.multiple_of(rr, _ROWS), _ROWS)
            cols = pl.ds(pl.multiple_of(cc, _COLS), _COLS)
            zb = zbuf[rows, cols]
            win = inb & inside & (z <= zb)
            zbuf[rows, cols] = jnp.where(win, z, zb)
            cb = out_ref[rows, cols]
            out_ref[rows, cols] = jnp.where(win, col, cb)
            cc2 = cc + _COLS
            wrap = cc2 >= c_hi
            return (jnp.where(wrap, rr + _ROWS, rr),
                    jnp.where(wrap, c_lo, cc2))

        lax.fori_loop(0, total, win_body, (r_lo, c_lo))
        return carry

    lax.fori_loop(0, _CHUNK, face_body, 0)


@jax.jit
def kernel(vertices, faces):
    n_faces = faces.shape[0]
    elev = 30.0 * np.pi / 180.0
    azim = 0.0
    rot_y = jnp.array([[np.cos(azim), 0.0, np.sin(azim)],
                       [0.0, 1.0, 0.0],
                       [-np.sin(azim), 0.0, np.cos(azim)]], dtype=jnp.float32)
    rot_x = jnp.array([[1.0, 0.0, 0.0],
                       [0.0, np.cos(elev), -np.sin(elev)],
                       [0.0, np.sin(elev), np.cos(elev)]], dtype=jnp.float32)
    vr = vertices @ (rot_y @ rot_x)
    vr = vr.at[:, 2].add(2.0)
    v2d = vr[:, :2] / vr[:, 2:3]
    v2d = (v2d + 1.0) * _S / 2.0
    xi = v2d[:, 0].astype(jnp.int32)
    yi = v2d[:, 1].astype(jnp.int32)
    vrx, vry, vrz = vr[:, 0], vr[:, 1], vr[:, 2]

    f_pad = jnp.zeros((_NPAD, 3), jnp.int32)
    f_pad = f_pad.at[:n_faces].set(faces.astype(jnp.int32))
    f0, f1, f2 = f_pad[:, 0], f_pad[:, 1], f_pad[:, 2]

    pi, pf = _sc_face_params(xi, yi, vrx, vry, vrz, f0, f1, f2)

    n_grid = _NPAD // _CHUNK
    tri_blocks = pi.reshape(16, n_grid, 1, _CHUNK)
    fpar_blocks = pf.reshape(2, n_grid, 1, _CHUNK)

    color = pl.pallas_call(
        _raster_body,
        grid=(n_grid,),
        in_specs=[
            pl.BlockSpec((16, 1, 1, _CHUNK), lambda g: (0, g, 0, 0),
                         memory_space=pltpu.SMEM),
            pl.BlockSpec((2, 1, 1, _CHUNK), lambda g: (0, g, 0, 0),
                         memory_space=pltpu.SMEM),
        ],
        out_specs=pl.BlockSpec((_FB, _FB), lambda g: (0, 0)),
        out_shape=jax.ShapeDtypeStruct((_FB, _FB), jnp.float32),
        scratch_shapes=[
            pltpu.VMEM((_FB, _FB), jnp.float32),
        ],
    )(tri_blocks, fpar_blocks)

    return jnp.broadcast_to(color[:_S, :_S, None], (_S, _S, 3))
